# Initial kernel scaffold; baseline (speedup 1.0000x reference)
import jax
import jax.numpy as jnp
from jax.experimental import pallas as pl


def kernel(x, task_ids, hash_indices, W1, b1, W2, b2, ln_w, ln_b):
    raise NotImplementedError("write your pallas kernel here")


# speedup vs baseline: 7.5611x; 7.5611x over previous
import jax
import jax.numpy as jnp
from jax.experimental import pallas as pl
from jax.experimental.pallas import tpu as pltpu

E = 8
HIDDEN = 1024
INTER = 4096
EPS = 1e-12
S, L = 32, 128
NI = 2
BI = INTER // NI


def _routing_schedule(task_ids, hash_indices):
    dest = hash_indices[task_ids, 0].astype(jnp.int32)
    r_ar = jnp.arange(S, dtype=jnp.int32)
    e_ar = jnp.arange(E, dtype=jnp.int32)
    oh = (dest[None, :] == e_ar[:, None])
    ohf = oh.astype(jnp.float32)
    ohT = oh.astype(jnp.int32).T
    rank = (jnp.cumsum(ohT, axis=0) - 1)[r_ar, dest]
    cnt = jnp.sum(ohT, axis=0)
    start = jnp.concatenate([jnp.zeros((1,), jnp.int32),
                             jnp.cumsum(cnt)[:-1].astype(jnp.int32)])
    first_tok = jnp.full((E,), S, jnp.int32).at[dest].min(r_ar)
    pos_sum = jnp.where(cnt > 0, start, S)
    pos_tok = jnp.where(rank > 0, start[dest] + rank, S)
    z = jnp.zeros((S,), jnp.int32)
    src = z.at[pos_sum].set(e_ar, mode="drop").at[pos_tok].set(E + r_ar, mode="drop")
    eid = z.at[pos_sum].set(e_ar, mode="drop").at[pos_tok].set(dest, mode="drop")
    issum = z.at[pos_sum].set(1, mode="drop")
    orow = z.at[pos_sum].set(first_tok, mode="drop").at[pos_tok].set(r_ar, mode="drop")
    return jnp.stack([src, eid, issum, orow]), ohf


def _sums_body(oh_ref, x_ref, o_ref):
    o_ref[...] = jax.lax.dot(oh_ref[...], x_ref[...],
                             precision=jax.lax.Precision.HIGHEST,
                             preferred_element_type=jnp.float32)


def _compute_sums(ohf, xf):
    CH = 8192
    return pl.pallas_call(
        _sums_body,
        grid=(xf.shape[1] // CH,),
        in_specs=[pl.BlockSpec((E, S), lambda c: (0, 0)),
                  pl.BlockSpec((S, CH), lambda c: (0, c))],
        out_specs=pl.BlockSpec((E, CH), lambda c: (0, c)),
        out_shape=jax.ShapeDtypeStruct((E, xf.shape[1]), jnp.float32),
    )(ohf, xf)


def _ffn_body(sched, sums_ref, x_ref, w1_ref, w2_ref, b1_ref, b2_ref,
              lnw_ref, lnb_ref, o_ref, acc, fsum):
    i = pl.program_id(0)
    j = pl.program_id(1)
    is_sum = sched[2, i] == 1
    xv = jnp.where(is_sum, sums_ref[0], x_ref[0])
    xb = xv.astype(jnp.bfloat16)
    w1 = w1_ref[0].astype(jnp.bfloat16)
    y = jnp.dot(xb, w1, preferred_element_type=jnp.float32) + b1_ref[0, 0]
    y = y * 0.5 * (1.0 + jax.lax.erf(y * (2.0 ** -0.5)))
    p = jnp.dot(y.astype(jnp.bfloat16), w2_ref[0].astype(jnp.bfloat16),
                preferred_element_type=jnp.float32)

    @pl.when(j == 0)
    def _():
        acc[...] = p

    @pl.when(j == NI - 1)
    def _():
        zv = acc[...] + p + b2_ref[0, 0] + xv
        mu = jnp.mean(zv, axis=-1, keepdims=True)
        d = zv - mu
        var = jnp.mean(d * d, axis=-1, keepdims=True)
        ln = d * jax.lax.rsqrt(var + EPS) * lnw_ref[0, 0] + lnb_ref[0, 0]
        prev = fsum[...]
        fsum[...] = jnp.where(is_sum, ln, prev)
        o_ref[0] = ln + jnp.where(is_sum, jnp.zeros_like(ln), prev)


def _jj(i, j):
    return jnp.where(i % 2 == 0, j, NI - 1 - j)


def _ffn(sched, sums, x, W1, W2, b1r, b2r, lnwr, lnbr):
    grid_spec = pltpu.PrefetchScalarGridSpec(
        num_scalar_prefetch=1,
        grid=(S, NI),
        in_specs=[
            pl.BlockSpec((1, L, HIDDEN),
                         lambda i, j, s: (jnp.minimum(s[0, i], E - 1), 0, 0)),
            pl.BlockSpec((1, L, HIDDEN),
                         lambda i, j, s: (jnp.clip(s[0, i] - E, 0, S - 1), 0, 0)),
            pl.BlockSpec((1, HIDDEN, BI),
                         lambda i, j, s: (s[1, i], 0, _jj(i, j))),
            pl.BlockSpec((1, BI, HIDDEN),
                         lambda i, j, s: (s[1, i], _jj(i, j), 0)),
            pl.BlockSpec((1, 1, BI),
                         lambda i, j, s: (s[1, i] * NI + _jj(i, j), 0, 0)),
            pl.BlockSpec((1, 1, HIDDEN), lambda i, j, s: (s[1, i], 0, 0)),
            pl.BlockSpec((1, 1, HIDDEN), lambda i, j, s: (s[1, i], 0, 0)),
            pl.BlockSpec((1, 1, HIDDEN), lambda i, j, s: (s[1, i], 0, 0)),
        ],
        out_specs=pl.BlockSpec((1, L, HIDDEN),
                               lambda i, j, s: (s[3, i], 0, 0)),
        scratch_shapes=[pltpu.VMEM((L, HIDDEN), jnp.float32),
                        pltpu.VMEM((L, HIDDEN), jnp.float32)],
    )
    return pl.pallas_call(
        _ffn_body,
        grid_spec=grid_spec,
        out_shape=jax.ShapeDtypeStruct((S, L, HIDDEN), jnp.float32),
    )(sched, sums, x, W1, W2, b1r, b2r, lnwr, lnbr)


def kernel(x, task_ids, hash_indices, W1, b1, W2, b2, ln_w, ln_b):
    sched, ohf = _routing_schedule(task_ids, hash_indices)
    sums = _compute_sums(ohf, x.reshape(S, L * HIDDEN)).reshape(E, L, HIDDEN)
    out = _ffn(sched, sums, x, W1, W2,
               b1.reshape(E * NI, 1, BI), b2.reshape(E, 1, HIDDEN),
               ln_w.reshape(E, 1, HIDDEN), ln_b.reshape(E, 1, HIDDEN))
    return (out, task_ids)


# speedup vs baseline: 7.7685x; 1.0274x over previous
import jax
import jax.numpy as jnp
from jax.experimental import pallas as pl
from jax.experimental.pallas import tpu as pltpu
from jax.experimental.pallas import tpu_sc as plsc

E = 8
HIDDEN = 1024
INTER = 4096
EPS = 1e-12
S, L = 32, 128
NI = 2
BI = INTER // NI


def _route_body(meta_hbm, sched_hbm, oh_hbm, m_s, dest_s, rank_s, cnt_s,
                start_s, ft_s, run_s, sched_s, oh_s, sem):
    @pl.when(jax.lax.axis_index("c") == 0)
    def _():
        pltpu.async_copy(meta_hbm, m_s, sem).wait()

        @pl.loop(0, E)
        def _(e):
            cnt_s[e] = 0
            ft_s[e] = 0

            @pl.loop(0, S)
            def _(r):
                oh_s[e, r] = 0.0

        @pl.loop(0, S)
        def _(r):
            d = m_s[S + m_s[r]]
            dest_s[r] = d
            c = cnt_s[d]
            rank_s[r] = c
            ft_s[d] = jnp.where(c == 0, r, ft_s[d])
            cnt_s[d] = c + 1
            oh_s[d, r] = 1.0

        run_s[0] = 0

        @pl.loop(0, E)
        def _(e):
            start_s[e] = run_s[0]
            run_s[0] = run_s[0] + cnt_s[e]

        @pl.loop(0, E)
        def _(e):
            @pl.when(cnt_s[e] > 0)
            def _():
                p = start_s[e]
                sched_s[0, p] = e
                sched_s[1, p] = e
                sched_s[2, p] = 1
                sched_s[3, p] = ft_s[e]

        @pl.loop(0, S)
        def _(r):
            @pl.when(rank_s[r] > 0)
            def _():
                d = dest_s[r]
                p = start_s[d] + rank_s[r]
                sched_s[0, p] = E + r
                sched_s[1, p] = d
                sched_s[2, p] = 0
                sched_s[3, p] = r

        pltpu.async_copy(sched_s, sched_hbm, sem).wait()
        pltpu.async_copy(oh_s, oh_hbm, sem).wait()


def _routing_schedule(task_ids, hash_indices):
    meta = jnp.concatenate([task_ids.astype(jnp.int32),
                            hash_indices[:, 0].astype(jnp.int32),
                            jnp.zeros((24,), jnp.int32)])
    route = pl.kernel(
        _route_body,
        out_type=[jax.ShapeDtypeStruct((4, S), jnp.int32),
                  jax.ShapeDtypeStruct((E, S), jnp.float32)],
        mesh=plsc.ScalarSubcoreMesh(axis_name="c", num_cores=2),
        scratch_types=[pltpu.SMEM((64,), jnp.int32),
                       pltpu.SMEM((S,), jnp.int32),
                       pltpu.SMEM((S,), jnp.int32),
                       pltpu.SMEM((E,), jnp.int32),
                       pltpu.SMEM((E,), jnp.int32),
                       pltpu.SMEM((E,), jnp.int32),
                       pltpu.SMEM((1,), jnp.int32),
                       pltpu.SMEM((4, S), jnp.int32),
                       pltpu.SMEM((E, S), jnp.float32),
                       pltpu.SemaphoreType.DMA],
    )
    return route(meta)


def _sums_body(oh_ref, x_ref, o_ref):
    o_ref[...] = jax.lax.dot(oh_ref[...], x_ref[...],
                             precision=jax.lax.Precision.HIGHEST,
                             preferred_element_type=jnp.float32)


def _compute_sums(ohf, xf):
    CH = 8192
    return pl.pallas_call(
        _sums_body,
        grid=(xf.shape[1] // CH,),
        in_specs=[pl.BlockSpec((E, S), lambda c: (0, 0)),
                  pl.BlockSpec((S, CH), lambda c: (0, c))],
        out_specs=pl.BlockSpec((E, CH), lambda c: (0, c)),
        out_shape=jax.ShapeDtypeStruct((E, xf.shape[1]), jnp.float32),
    )(ohf, xf)


def _ffn_body(sched, sums_ref, x_ref, w1_ref, w2_ref, b1_ref, b2_ref,
              lnw_ref, lnb_ref, o_ref, acc, fsum):
    i = pl.program_id(0)
    j = pl.program_id(1)
    is_sum = sched[2, i] == 1
    xv = jnp.where(is_sum, sums_ref[0], x_ref[0])
    xb = xv.astype(jnp.bfloat16)
    w1 = w1_ref[0].astype(jnp.bfloat16)
    y = jnp.dot(xb, w1, preferred_element_type=jnp.float32) + b1_ref[0, 0]
    y = y * 0.5 * (1.0 + jax.lax.erf(y * (2.0 ** -0.5)))
    p = jnp.dot(y.astype(jnp.bfloat16), w2_ref[0].astype(jnp.bfloat16),
                preferred_element_type=jnp.float32)

    @pl.when(j == 0)
    def _():
        acc[...] = p

    @pl.when(j == NI - 1)
    def _():
        zv = acc[...] + p + b2_ref[0, 0] + xv
        mu = jnp.mean(zv, axis=-1, keepdims=True)
        d = zv - mu
        var = jnp.mean(d * d, axis=-1, keepdims=True)
        ln = d * jax.lax.rsqrt(var + EPS) * lnw_ref[0, 0] + lnb_ref[0, 0]
        prev = fsum[...]
        fsum[...] = jnp.where(is_sum, ln, prev)
        o_ref[0] = ln + jnp.where(is_sum, jnp.zeros_like(ln), prev)


def _jj(i, j):
    return jnp.where(i % 2 == 0, j, NI - 1 - j)


def _ffn(sched, sums, x, W1, W2, b1r, b2r, lnwr, lnbr):
    grid_spec = pltpu.PrefetchScalarGridSpec(
        num_scalar_prefetch=1,
        grid=(S, NI),
        in_specs=[
            pl.BlockSpec((1, L, HIDDEN),
                         lambda i, j, s: (jnp.minimum(s[0, i], E - 1), 0, 0)),
            pl.BlockSpec((1, L, HIDDEN),
                         lambda i, j, s: (jnp.clip(s[0, i] - E, 0, S - 1), 0, 0)),
            pl.BlockSpec((1, HIDDEN, BI),
                         lambda i, j, s: (s[1, i], 0, _jj(i, j))),
            pl.BlockSpec((1, BI, HIDDEN),
                         lambda i, j, s: (s[1, i], _jj(i, j), 0)),
            pl.BlockSpec((1, 1, BI),
                         lambda i, j, s: (s[1, i] * NI + _jj(i, j), 0, 0)),
            pl.BlockSpec((1, 1, HIDDEN), lambda i, j, s: (s[1, i], 0, 0)),
            pl.BlockSpec((1, 1, HIDDEN), lambda i, j, s: (s[1, i], 0, 0)),
            pl.BlockSpec((1, 1, HIDDEN), lambda i, j, s: (s[1, i], 0, 0)),
        ],
        out_specs=pl.BlockSpec((1, L, HIDDEN),
                               lambda i, j, s: (s[3, i], 0, 0)),
        scratch_shapes=[pltpu.VMEM((L, HIDDEN), jnp.float32),
                        pltpu.VMEM((L, HIDDEN), jnp.float32)],
    )
    return pl.pallas_call(
        _ffn_body,
        grid_spec=grid_spec,
        out_shape=jax.ShapeDtypeStruct((S, L, HIDDEN), jnp.float32),
    )(sched, sums, x, W1, W2, b1r, b2r, lnwr, lnbr)


def kernel(x, task_ids, hash_indices, W1, b1, W2, b2, ln_w, ln_b):
    sched, ohf = _routing_schedule(task_ids, hash_indices)
    sums = _compute_sums(ohf, x.reshape(S, L * HIDDEN)).reshape(E, L, HIDDEN)
    out = _ffn(sched, sums, x, W1, W2,
               b1.reshape(E * NI, 1, BI), b2.reshape(E, 1, HIDDEN),
               ln_w.reshape(E, 1, HIDDEN), ln_b.reshape(E, 1, HIDDEN))
    return (out, task_ids)


# speedup vs baseline: 9.1908x; 1.1831x over previous
import jax
import jax.numpy as jnp
from jax.experimental import pallas as pl
from jax.experimental.pallas import tpu as pltpu
from jax.experimental.pallas import tpu_sc as plsc

E = 8
HIDDEN = 1024
INTER = 4096
EPS = 1e-12
S, L = 32, 128
NI = 8
BI = INTER // NI


def _route_body(meta_hbm, sched_hbm, oh_hbm, m_s, dest_s, rank_s, cnt_s,
                start_s, ft_s, run_s, sched_s, oh_s, sem):
    @pl.when(jax.lax.axis_index("c") == 0)
    def _():
        pltpu.async_copy(meta_hbm, m_s, sem).wait()

        @pl.loop(0, E)
        def _(e):
            cnt_s[e] = 0
            ft_s[e] = 0

            @pl.loop(0, S)
            def _(r):
                oh_s[e, r] = 0.0

        @pl.loop(0, S)
        def _(r):
            d = m_s[S + m_s[r]]
            dest_s[r] = d
            c = cnt_s[d]
            rank_s[r] = c
            ft_s[d] = jnp.where(c == 0, r, ft_s[d])
            cnt_s[d] = c + 1
            oh_s[d, r] = 1.0

        run_s[0] = 0

        @pl.loop(0, E)
        def _(e):
            start_s[e] = run_s[0]
            run_s[0] = run_s[0] + cnt_s[e]

        @pl.loop(0, S)
        def _(r):
            sched_s[4, r] = 0
            sched_s[5, r] = 0

        @pl.loop(0, E)
        def _(e):
            sched_s[4, e] = start_s[e]
            sched_s[5, e] = cnt_s[e]

            @pl.when(cnt_s[e] > 0)
            def _():
                p = start_s[e]
                sched_s[0, p] = e
                sched_s[1, p] = e
                sched_s[2, p] = 1
                sched_s[3, p] = ft_s[e]

        @pl.loop(0, S)
        def _(r):
            @pl.when(rank_s[r] > 0)
            def _():
                d = dest_s[r]
                p = start_s[d] + rank_s[r]
                sched_s[0, p] = E + r
                sched_s[1, p] = d
                sched_s[2, p] = 0
                sched_s[3, p] = r

        pltpu.async_copy(sched_s, sched_hbm, sem).wait()
        pltpu.async_copy(oh_s, oh_hbm, sem).wait()


def _routing_schedule(task_ids, hash_indices):
    meta = jnp.concatenate([task_ids.astype(jnp.int32),
                            hash_indices[:, 0].astype(jnp.int32),
                            jnp.zeros((24,), jnp.int32)])
    route = pl.kernel(
        _route_body,
        out_type=[jax.ShapeDtypeStruct((6, S), jnp.int32),
                  jax.ShapeDtypeStruct((E, S), jnp.float32)],
        mesh=plsc.ScalarSubcoreMesh(axis_name="c", num_cores=2),
        scratch_types=[pltpu.SMEM((64,), jnp.int32),
                       pltpu.SMEM((S,), jnp.int32),
                       pltpu.SMEM((S,), jnp.int32),
                       pltpu.SMEM((E,), jnp.int32),
                       pltpu.SMEM((E,), jnp.int32),
                       pltpu.SMEM((E,), jnp.int32),
                       pltpu.SMEM((1,), jnp.int32),
                       pltpu.SMEM((6, S), jnp.int32),
                       pltpu.SMEM((E, S), jnp.float32),
                       pltpu.SemaphoreType.DMA],
    )
    return route(meta)


def _sums_body(oh_ref, x_ref, o_ref):
    o_ref[...] = jax.lax.dot(oh_ref[...], x_ref[...],
                             precision=jax.lax.Precision.HIGHEST,
                             preferred_element_type=jnp.float32)


def _compute_sums(ohf, xf):
    CH = 8192
    return pl.pallas_call(
        _sums_body,
        grid=(xf.shape[1] // CH,),
        in_specs=[pl.BlockSpec((E, S), lambda c: (0, 0)),
                  pl.BlockSpec((S, CH), lambda c: (0, c))],
        out_specs=pl.BlockSpec((E, CH), lambda c: (0, c)),
        out_shape=jax.ShapeDtypeStruct((E, xf.shape[1]), jnp.float32),
    )(ohf, xf)


def _ffn_body(sched, sums_ref, x_ref, w1_ref, w2_ref, b1_ref, b2_ref,
              lnw_ref, lnb_ref, o_hbm, acc_all, fsum, w1c, w2c, ostage, sem):
    e = pl.program_id(0)
    j = pl.program_id(1)
    start = sched[4, e]
    cnt = sched[5, e]
    w1c[...] = w1_ref[0].astype(jnp.bfloat16)
    w2c[...] = w2_ref[0].astype(jnp.bfloat16)

    def item(k, carry):
        p = start + k
        src = sched[0, p]
        is_sum = sched[2, p] == 1
        xv = jnp.where(is_sum,
                       sums_ref[jnp.minimum(src, E - 1)],
                       x_ref[jnp.clip(src - E, 0, S - 1)])
        xb = xv.astype(jnp.bfloat16)
        y = jnp.dot(xb, w1c[...], preferred_element_type=jnp.float32) + b1_ref[0, 0]
        y = y * 0.5 * (1.0 + jax.lax.erf(y * (2.0 ** -0.5)))
        pk = jnp.dot(y.astype(jnp.bfloat16), w2c[...],
                     preferred_element_type=jnp.float32)
        prev = jnp.where(j == 0, jnp.zeros_like(pk), acc_all[k])
        tot = prev + pk

        @pl.when(j < NI - 1)
        def _():
            acc_all[k] = tot

        @pl.when(j == NI - 1)
        def _():
            zv = tot + b2_ref[0, 0] + xv
            mu = jnp.mean(zv, axis=-1, keepdims=True)
            d = zv - mu
            var = jnp.mean(d * d, axis=-1, keepdims=True)
            ln = d * jax.lax.rsqrt(var + EPS) * lnw_ref[0, 0] + lnb_ref[0, 0]
            prev_fs = fsum[...]
            fsum[...] = jnp.where(is_sum, ln, prev_fs)
            outv = ln + jnp.where(is_sum, jnp.zeros_like(ln), prev_fs)
            slot = jax.lax.rem(p, 2)
            orow = sched[3, p]

            @pl.when(p >= 2)
            def _():
                pltpu.make_async_copy(ostage.at[slot], o_hbm.at[orow],
                                      sem.at[slot]).wait()

            ostage[slot] = outv
            pltpu.make_async_copy(ostage.at[slot], o_hbm.at[orow],
                                  sem.at[slot]).start()

        return carry

    jax.lax.fori_loop(0, cnt, item, 0)

    @pl.when((e == E - 1) & (j == NI - 1))
    def _():
        pltpu.make_async_copy(ostage.at[0], o_hbm.at[0], sem.at[0]).wait()
        pltpu.make_async_copy(ostage.at[1], o_hbm.at[0], sem.at[1]).wait()


def _ffn(sched, sums, x, W1, W2, b1r, b2r, lnwr, lnbr):
    grid_spec = pltpu.PrefetchScalarGridSpec(
        num_scalar_prefetch=1,
        grid=(E, NI),
        in_specs=[
            pl.BlockSpec((E, L, HIDDEN), lambda e, j, s: (0, 0, 0)),
            pl.BlockSpec((S, L, HIDDEN), lambda e, j, s: (0, 0, 0)),
            pl.BlockSpec((1, HIDDEN, BI), lambda e, j, s: (e, 0, j)),
            pl.BlockSpec((1, BI, HIDDEN), lambda e, j, s: (e, j, 0)),
            pl.BlockSpec((1, 1, BI), lambda e, j, s: (e * NI + j, 0, 0)),
            pl.BlockSpec((1, 1, HIDDEN), lambda e, j, s: (e, 0, 0)),
            pl.BlockSpec((1, 1, HIDDEN), lambda e, j, s: (e, 0, 0)),
            pl.BlockSpec((1, 1, HIDDEN), lambda e, j, s: (e, 0, 0)),
        ],
        out_specs=pl.BlockSpec(memory_space=pl.ANY),
        scratch_shapes=[pltpu.VMEM((S, L, HIDDEN), jnp.float32),
                        pltpu.VMEM((L, HIDDEN), jnp.float32),
                        pltpu.VMEM((HIDDEN, BI), jnp.bfloat16),
                        pltpu.VMEM((BI, HIDDEN), jnp.bfloat16),
                        pltpu.VMEM((2, L, HIDDEN), jnp.float32),
                        pltpu.SemaphoreType.DMA((2,))],
    )
    return pl.pallas_call(
        _ffn_body,
        grid_spec=grid_spec,
        out_shape=jax.ShapeDtypeStruct((S, L, HIDDEN), jnp.float32),
    )(sched, sums, x, W1, W2, b1r, b2r, lnwr, lnbr)


def kernel(x, task_ids, hash_indices, W1, b1, W2, b2, ln_w, ln_b):
    sched, ohf = _routing_schedule(task_ids, hash_indices)
    sums = _compute_sums(ohf, x.reshape(S, L * HIDDEN)).reshape(E, L, HIDDEN)
    out = _ffn(sched, sums, x, W1, W2,
               b1.reshape(E * NI, 1, BI), b2.reshape(E, 1, HIDDEN),
               ln_w.reshape(E, 1, HIDDEN), ln_b.reshape(E, 1, HIDDEN))
    return (out, task_ids)


# speedup vs baseline: 10.6096x; 1.1544x over previous
import jax
import jax.numpy as jnp
from jax.experimental import pallas as pl
from jax.experimental.pallas import tpu as pltpu
from jax.experimental.pallas import tpu_sc as plsc

E = 8
HIDDEN = 1024
INTER = 4096
EPS = 1e-12
S, L = 32, 128
NI = 4
BI = INTER // NI


def _route_body(meta_hbm, sched_hbm, dm_hbm, m_s, dest_s, rank_s, cnt_s,
                start_s, ft_s, run_s, sched_s, dm_s, sem):
    @pl.when(jax.lax.axis_index("c") == 0)
    def _():
        pltpu.async_copy(meta_hbm, m_s, sem).wait()

        @pl.loop(0, E)
        def _(e):
            cnt_s[e] = 0
            ft_s[e] = 0

        @pl.loop(0, S)
        def _(r):
            d = m_s[S + m_s[r]]
            dest_s[r] = d
            c = cnt_s[d]
            rank_s[r] = c
            ft_s[d] = jnp.where(c == 0, r, ft_s[d])
            cnt_s[d] = c + 1

            @pl.loop(0, S)
            def _(q):
                dm_s[r, q] = 0.0

        run_s[0] = 0

        @pl.loop(0, E)
        def _(e):
            start_s[e] = run_s[0]
            run_s[0] = run_s[0] + cnt_s[e]

        @pl.loop(0, S)
        def _(r):
            sched_s[4, r] = 0
            sched_s[5, r] = 0

        @pl.loop(0, E)
        def _(e):
            sched_s[4, e] = start_s[e]
            sched_s[5, e] = cnt_s[e]

            @pl.when(cnt_s[e] > 0)
            def _():
                p = start_s[e]
                sched_s[0, p] = e
                sched_s[1, p] = e
                sched_s[2, p] = 1
                sched_s[3, p] = ft_s[e]

        @pl.loop(0, S)
        def _(r):
            d = dest_s[r]
            dm_s[start_s[d], r] = 1.0

            @pl.when(rank_s[r] > 0)
            def _():
                p = start_s[d] + rank_s[r]
                sched_s[0, p] = E + r
                sched_s[1, p] = d
                sched_s[2, p] = 0
                sched_s[3, p] = r
                dm_s[p, r] = 1.0

        pltpu.async_copy(sched_s, sched_hbm, sem).wait()
        pltpu.async_copy(dm_s, dm_hbm, sem).wait()


def _routing_schedule(task_ids, hash_indices):
    meta = jnp.concatenate([task_ids.astype(jnp.int32),
                            hash_indices[:, 0].astype(jnp.int32),
                            jnp.zeros((24,), jnp.int32)])
    route = pl.kernel(
        _route_body,
        out_type=[jax.ShapeDtypeStruct((6, S), jnp.int32),
                  jax.ShapeDtypeStruct((S, S), jnp.float32)],
        mesh=plsc.ScalarSubcoreMesh(axis_name="c", num_cores=2),
        scratch_types=[pltpu.SMEM((64,), jnp.int32),
                       pltpu.SMEM((S,), jnp.int32),
                       pltpu.SMEM((S,), jnp.int32),
                       pltpu.SMEM((E,), jnp.int32),
                       pltpu.SMEM((E,), jnp.int32),
                       pltpu.SMEM((E,), jnp.int32),
                       pltpu.SMEM((1,), jnp.int32),
                       pltpu.SMEM((6, S), jnp.int32),
                       pltpu.SMEM((S, S), jnp.float32),
                       pltpu.SemaphoreType.DMA],
    )
    return route(meta)


def _gather_body(dm_ref, x_ref, o_ref):
    o_ref[...] = jax.lax.dot(dm_ref[...], x_ref[...],
                             precision=jax.lax.Precision.HIGHEST,
                             preferred_element_type=jnp.float32)


def _dispatch_gather(dm, xf):
    CH = 16384
    return pl.pallas_call(
        _gather_body,
        grid=(xf.shape[1] // CH,),
        in_specs=[pl.BlockSpec((S, S), lambda c: (0, 0)),
                  pl.BlockSpec((S, CH), lambda c: (0, c))],
        out_specs=pl.BlockSpec((S, CH), lambda c: (0, c)),
        out_shape=jax.ShapeDtypeStruct((S, xf.shape[1]), jnp.float32),
    )(dm, xf)


def _ffn_body(sched, x_ref, w1_ref, w2_ref, b1_ref, b2_ref,
              lnw_ref, lnb_ref, o_hbm, fsum, w1c, w2c, xbc, ostage, sem):
    e = pl.program_id(0)
    j = pl.program_id(1)
    w1c[j] = w1_ref[0].astype(jnp.bfloat16)
    w2c[j] = w2_ref[0].astype(jnp.bfloat16)

    @pl.when(j == NI - 1)
    def _():
        start = sched[4, e]
        cnt = sched[5, e]

        def item(k, carry):
            p = start + k
            is_sum = sched[2, p] == 1
            xbc[...] = x_ref[p].astype(jnp.bfloat16)
            acc = None
            for jj in range(NI):
                y = jnp.dot(xbc[...], w1c[jj],
                            preferred_element_type=jnp.float32)
                y = y + b1_ref[0, 0, jj * BI:(jj + 1) * BI]
                y = y * 0.5 * (1.0 + jax.lax.erf(y * (2.0 ** -0.5)))
                pk = jnp.dot(y.astype(jnp.bfloat16), w2c[jj],
                             preferred_element_type=jnp.float32)
                acc = pk if acc is None else acc + pk
            zv = acc + b2_ref[0, 0] + x_ref[p]
            mu = jnp.mean(zv, axis=-1, keepdims=True)
            d = zv - mu
            var = jnp.mean(d * d, axis=-1, keepdims=True)
            ln = d * jax.lax.rsqrt(var + EPS) * lnw_ref[0, 0] + lnb_ref[0, 0]
            prev_fs = fsum[...]
            fsum[...] = jnp.where(is_sum, ln, prev_fs)
            outv = ln + jnp.where(is_sum, jnp.zeros_like(ln), prev_fs)
            slot = jax.lax.rem(p, 2)
            orow = sched[3, p]

            @pl.when(p >= 2)
            def _():
                pltpu.make_async_copy(ostage.at[slot], o_hbm.at[orow],
                                      sem.at[slot]).wait()

            ostage[slot] = outv
            pltpu.make_async_copy(ostage.at[slot], o_hbm.at[orow],
                                  sem.at[slot]).start()
            return carry

        jax.lax.fori_loop(0, cnt, item, 0)

        @pl.when(e == E - 1)
        def _():
            pltpu.make_async_copy(ostage.at[0], o_hbm.at[0], sem.at[0]).wait()
            pltpu.make_async_copy(ostage.at[1], o_hbm.at[0], sem.at[1]).wait()


def _ffn(sched, xs, W1, W2, b1r, b2r, lnwr, lnbr):
    grid_spec = pltpu.PrefetchScalarGridSpec(
        num_scalar_prefetch=1,
        grid=(E, NI),
        in_specs=[
            pl.BlockSpec((S, L, HIDDEN), lambda e, j, s: (0, 0, 0)),
            pl.BlockSpec((1, HIDDEN, BI), lambda e, j, s: (e, 0, j)),
            pl.BlockSpec((1, BI, HIDDEN), lambda e, j, s: (e, j, 0)),
            pl.BlockSpec((1, 1, INTER), lambda e, j, s: (e, 0, 0)),
            pl.BlockSpec((1, 1, HIDDEN), lambda e, j, s: (e, 0, 0)),
            pl.BlockSpec((1, 1, HIDDEN), lambda e, j, s: (e, 0, 0)),
            pl.BlockSpec((1, 1, HIDDEN), lambda e, j, s: (e, 0, 0)),
        ],
        out_specs=pl.BlockSpec(memory_space=pl.ANY),
        scratch_shapes=[pltpu.VMEM((L, HIDDEN), jnp.float32),
                        pltpu.VMEM((NI, HIDDEN, BI), jnp.bfloat16),
                        pltpu.VMEM((NI, BI, HIDDEN), jnp.bfloat16),
                        pltpu.VMEM((L, HIDDEN), jnp.bfloat16),
                        pltpu.VMEM((2, L, HIDDEN), jnp.float32),
                        pltpu.SemaphoreType.DMA((2,))],
    )
    return pl.pallas_call(
        _ffn_body,
        grid_spec=grid_spec,
        out_shape=jax.ShapeDtypeStruct((S, L, HIDDEN), jnp.float32),
    )(sched, xs, W1, W2, b1r, b2r, lnwr, lnbr)


def kernel(x, task_ids, hash_indices, W1, b1, W2, b2, ln_w, ln_b):
    sched, dm = _routing_schedule(task_ids, hash_indices)
    xs = _dispatch_gather(dm, x.reshape(S, L * HIDDEN)).reshape(S, L, HIDDEN)
    out = _ffn(sched, xs, W1, W2,
               b1.reshape(E, 1, INTER), b2.reshape(E, 1, HIDDEN),
               ln_w.reshape(E, 1, HIDDEN), ln_b.reshape(E, 1, HIDDEN))
    return (out, task_ids)
